# baseline (device time: 166837 ns/iter reference)
import jax
import jax.numpy as jnp
from jax import lax
from jax.experimental import pallas as pl
from jax.experimental.pallas import tpu as pltpu

N_DEV = 32
B, SQ, D = 2, 256, 768
H_LOC, DH = 8, 64
DL = H_LOC * DH
R = B * SQ
CH = R // N_DEV


def kernel(x, Wq, Wo, Wk, Wv):
    def body(x_ref, wq_ref, wo_ref, wk_ref, wv_ref, out_ref,
             acc_ref, comm_ref, rs_send, rs_recv, ag_send, ag_recv):
        my = lax.axis_index("i")
        right = lax.rem(my + 1, N_DEV)
        left = lax.rem(my + N_DEV - 1, N_DEV)

        xf = x_ref[...].reshape(R, D)
        q = jnp.dot(xf, wq_ref[...], preferred_element_type=jnp.float32)
        k = jnp.dot(xf, wk_ref[...], preferred_element_type=jnp.float32)
        v = jnp.dot(xf, wv_ref[...], preferred_element_type=jnp.float32)

        bouts = []
        for b in range(B):
            rows = slice(b * SQ, (b + 1) * SQ)
            houts = []
            for h in range(H_LOC):
                cols = slice(h * DH, (h + 1) * DH)
                qb, kb, vb = q[rows, cols], k[rows, cols], v[rows, cols]
                s = lax.dot_general(
                    qb, kb, (((1,), (1,)), ((), ())),
                    preferred_element_type=jnp.float32,
                ) * 0.125
                m = jnp.max(s, axis=-1, keepdims=True)
                p = jnp.exp(s - m)
                l = jnp.sum(p, axis=-1, keepdims=True)
                o = jnp.dot(p, vb, preferred_element_type=jnp.float32) / l
                houts.append(o)
            bouts.append(jnp.concatenate(houts, axis=1))
        attn = jnp.concatenate(bouts, axis=0)
        acc_ref[...] = jnp.dot(attn, wo_ref[...],
                               preferred_element_type=jnp.float32)

        barrier = pltpu.get_barrier_semaphore()
        for nbr in (left, right):
            pl.semaphore_signal(barrier, inc=1, device_id=(nbr,),
                                device_id_type=pl.DeviceIdType.MESH)
        pl.semaphore_wait(barrier, 2)

        for s_ in range(N_DEV - 1):
            src_c = lax.rem(my - s_ + N_DEV, N_DEV)
            rdma = pltpu.make_async_remote_copy(
                src_ref=acc_ref.at[pl.ds(src_c * CH, CH), :],
                dst_ref=comm_ref.at[pl.ds(s_ * CH, CH), :],
                send_sem=rs_send.at[s_],
                recv_sem=rs_recv.at[s_],
                device_id=(right,),
                device_id_type=pl.DeviceIdType.MESH,
            )
            rdma.start()
            rdma.wait()
            rc = lax.rem(my - s_ - 1 + 2 * N_DEV, N_DEV)
            acc_ref[pl.ds(rc * CH, CH), :] = (
                acc_ref[pl.ds(rc * CH, CH), :]
                + comm_ref[pl.ds(s_ * CH, CH), :]
            )

        for s_ in range(N_DEV - 1):
            sc = lax.rem(my + 1 - s_ + N_DEV, N_DEV)
            rdma = pltpu.make_async_remote_copy(
                src_ref=acc_ref.at[pl.ds(sc * CH, CH), :],
                dst_ref=acc_ref.at[pl.ds(sc * CH, CH), :],
                send_sem=ag_send.at[s_],
                recv_sem=ag_recv.at[s_],
                device_id=(right,),
                device_id_type=pl.DeviceIdType.MESH,
            )
            rdma.start()
            rdma.wait()

        out_ref[...] = acc_ref[...].reshape(B, SQ, D)

    return pl.pallas_call(
        body,
        out_shape=jax.ShapeDtypeStruct((B, SQ, D), jnp.float32),
        in_specs=[pl.BlockSpec(memory_space=pltpu.VMEM)] * 5,
        out_specs=pl.BlockSpec(memory_space=pltpu.VMEM),
        scratch_shapes=[
            pltpu.VMEM((R, D), jnp.float32),
            pltpu.VMEM(((N_DEV - 1) * CH, D), jnp.float32),
            pltpu.SemaphoreType.DMA((N_DEV - 1,)),
            pltpu.SemaphoreType.DMA((N_DEV - 1,)),
            pltpu.SemaphoreType.DMA((N_DEV - 1,)),
            pltpu.SemaphoreType.DMA((N_DEV - 1,)),
        ],
        compiler_params=pltpu.CompilerParams(collective_id=0),
    )(x, Wq, Wo, Wk, Wv)


# device time: 71518 ns/iter; 2.3328x vs baseline; 2.3328x over previous
import jax
import jax.numpy as jnp
from jax import lax
from jax.experimental import pallas as pl
from jax.experimental.pallas import tpu as pltpu

N_DEV = 32
LOG_N = 5
B, SQ, D = 2, 256, 768
H_LOC, DH = 8, 64
R = B * SQ

HALVES = [R >> (k + 1) for k in range(LOG_N)]
OFFS = [sum(HALVES[:k]) for k in range(LOG_N)]
COMM_ROWS = sum(HALVES)


def kernel(x, Wq, Wo, Wk, Wv):
    def body(x_ref, wq_ref, wo_ref, wk_ref, wv_ref, out_ref,
             acc_ref, comm_ref, rs_send, rs_recv, ag_send, ag_recv):
        my = lax.axis_index("i")

        xf = x_ref[...].reshape(R, D)
        q = jnp.dot(xf, wq_ref[...], preferred_element_type=jnp.float32)
        k = jnp.dot(xf, wk_ref[...], preferred_element_type=jnp.float32)
        v = jnp.dot(xf, wv_ref[...], preferred_element_type=jnp.float32)

        bouts = []
        for b in range(B):
            rows = slice(b * SQ, (b + 1) * SQ)
            houts = []
            for h in range(H_LOC):
                cols = slice(h * DH, (h + 1) * DH)
                qb, kb, vb = q[rows, cols], k[rows, cols], v[rows, cols]
                s = lax.dot_general(
                    qb, kb, (((1,), (1,)), ((), ())),
                    preferred_element_type=jnp.float32,
                ) * 0.125
                m = jnp.max(s, axis=-1, keepdims=True)
                p = jnp.exp(s - m)
                l = jnp.sum(p, axis=-1, keepdims=True)
                o = jnp.dot(p, vb, preferred_element_type=jnp.float32) / l
                houts.append(o)
            bouts.append(jnp.concatenate(houts, axis=1))
        attn = jnp.concatenate(bouts, axis=0)
        acc_ref[...] = jnp.dot(attn, wo_ref[...],
                               preferred_element_type=jnp.float32)

        barrier = pltpu.get_barrier_semaphore()
        for kk in range(LOG_N):
            pl.semaphore_signal(barrier, inc=1, device_id=(my ^ (1 << kk),),
                                device_id_type=pl.DeviceIdType.MESH)
        pl.semaphore_wait(barrier, LOG_N)

        lo = my * 0
        for kk in range(LOG_N):
            half = HALVES[kk]
            bit = (my >> kk) & 1
            partner = my ^ (1 << kk)
            send_lo = lo + (1 - bit) * half
            keep_lo = lo + bit * half
            rdma = pltpu.make_async_remote_copy(
                src_ref=acc_ref.at[pl.ds(pl.multiple_of(send_lo, 16), half), :],
                dst_ref=comm_ref.at[pl.ds(OFFS[kk], half), :],
                send_sem=rs_send.at[kk],
                recv_sem=rs_recv.at[kk],
                device_id=(partner,),
                device_id_type=pl.DeviceIdType.MESH,
            )
            rdma.start()
            rdma.wait()
            acc_ref[pl.ds(pl.multiple_of(keep_lo, 16), half), :] = (
                acc_ref[pl.ds(pl.multiple_of(keep_lo, 16), half), :]
                + comm_ref[OFFS[kk]:OFFS[kk] + half, :]
            )
            lo = keep_lo

        for kk in reversed(range(LOG_N)):
            sz = HALVES[kk]
            bit = (my >> kk) & 1
            partner = my ^ (1 << kk)
            plo = lo + (1 - 2 * bit) * sz
            rdma = pltpu.make_async_remote_copy(
                src_ref=acc_ref.at[pl.ds(pl.multiple_of(lo, 16), sz), :],
                dst_ref=acc_ref.at[pl.ds(pl.multiple_of(lo, 16), sz), :],
                send_sem=ag_send.at[kk],
                recv_sem=ag_recv.at[kk],
                device_id=(partner,),
                device_id_type=pl.DeviceIdType.MESH,
            )
            rdma.start()
            rdma.wait()
            lo = jnp.minimum(lo, plo)

        out_ref[...] = acc_ref[...].reshape(B, SQ, D)

    return pl.pallas_call(
        body,
        out_shape=jax.ShapeDtypeStruct((B, SQ, D), jnp.float32),
        in_specs=[pl.BlockSpec(memory_space=pltpu.VMEM)] * 5,
        out_specs=pl.BlockSpec(memory_space=pltpu.VMEM),
        scratch_shapes=[
            pltpu.VMEM((R, D), jnp.float32),
            pltpu.VMEM((COMM_ROWS, D), jnp.float32),
            pltpu.SemaphoreType.DMA((LOG_N,)),
            pltpu.SemaphoreType.DMA((LOG_N,)),
            pltpu.SemaphoreType.DMA((LOG_N,)),
            pltpu.SemaphoreType.DMA((LOG_N,)),
        ],
        compiler_params=pltpu.CompilerParams(collective_id=0),
    )(x, Wq, Wo, Wk, Wv)
